# Initial kernel scaffold; baseline (speedup 1.0000x reference)
import jax
import jax.numpy as jnp
from jax.experimental import pallas as pl


def kernel(x, edge_index, W1, b1, W2, b2):
    raise NotImplementedError("write your pallas kernel here")


# speedup vs baseline: 6.8453x; 6.8453x over previous
import functools

import jax
import jax.numpy as jnp
from jax import lax
from jax.experimental import pallas as pl
from jax.experimental.pallas import tpu as pltpu
from jax.experimental.pallas import tpu_sc as plsc

N = 10000
D = 128
E = 320000
NC = 2
NS = 16
NW = NC * NS
CE = 128
CH = 80
EPT = CH * CE
EPAD = NW * EPT
NPAD = 10240
RPT = NPAD // NS
TRASH = N


def _mesh():
    return plsc.VectorSubcoreMesh(core_axis_name="c", subcore_axis_name="s")


def _deg_body(row_hbm, col_hbm, out_hbm, row_v, col_v, ridx_v, ones_v, buf_v,
              hist_sh):
    c = lax.axis_index("c")
    s = lax.axis_index("s")
    w = s * NC + c
    pltpu.sync_copy(row_hbm.at[w], row_v)
    pltpu.sync_copy(col_hbm.at[w], col_v)
    one = jnp.ones((16,), jnp.float32)
    zero = jnp.zeros((16,), jnp.float32)
    for k in range(CE // 16):
        ones_v[pl.ds(k * 16, 16)] = one

    def _remap(j, carry):
        for k in range(CE // 16):
            sl = pl.ds(k * 16, 16)
            r = row_v[j, sl]
            cc = col_v[j, sl]
            ridx_v[j, sl] = jnp.where(r == cc, TRASH, r)
        return carry

    lax.fori_loop(0, CH, _remap, 0)

    def _zb(i, carry):
        buf_v[pl.ds(i * 16, 16)] = zero
        return carry

    lax.fori_loop(0, RPT // 16, _zb, 0)
    pltpu.sync_copy(buf_v, hist_sh.at[pl.ds(s * RPT, RPT)])
    plsc.subcore_barrier()

    def _acc(j, carry):
        pltpu.sync_copy(ones_v, hist_sh.at[ridx_v.at[j]], add=True)
        return carry

    lax.fori_loop(0, CH, _acc, 0)
    plsc.subcore_barrier()
    pltpu.sync_copy(hist_sh.at[pl.ds(s * RPT, RPT)], buf_v)
    pltpu.sync_copy(buf_v, out_hbm.at[c, pl.ds(s * RPT, RPT)])


def _prop_body(g_hbm, row_hbm, col_hbm, out_hbm, row_v, col_v, cfx_v, rows_v,
               acc_sh, sem):
    c = lax.axis_index("c")
    s = lax.axis_index("s")
    w = s * NC + c
    pltpu.sync_copy(row_hbm.at[w], row_v)
    pltpu.sync_copy(col_hbm.at[w], col_v)

    def _remap(j, carry):
        for k in range(CE // 16):
            sl = pl.ds(k * 16, 16)
            r = row_v[j, sl]
            cc = col_v[j, sl]
            cfx_v[j, sl] = jnp.where(r == cc, TRASH, cc)
        return carry

    lax.fori_loop(0, CH, _remap, 0)

    zero = jnp.zeros((16,), jnp.float32)

    def _zb(i, carry):
        for k in range(D // 16):
            rows_v[i, pl.ds(k * 16, 16)] = zero
        return carry

    lax.fori_loop(0, CE, _zb, 0)
    for t in range(RPT // CE):
        pltpu.sync_copy(rows_v, acc_sh.at[pl.ds(s * RPT + t * CE, CE)])
    plsc.subcore_barrier()

    def _step(j, carry):
        pltpu.async_copy(g_hbm.at[row_v.at[j]], rows_v, sem).wait()
        pltpu.sync_copy(rows_v, acc_sh.at[cfx_v.at[j]], add=True)
        return carry

    lax.fori_loop(0, CH, _step, 0)
    plsc.subcore_barrier()
    for t in range(RPT // CE):
        sl = pl.ds(s * RPT + t * CE, CE)
        pltpu.sync_copy(acc_sh.at[sl], rows_v)
        pltpu.sync_copy(rows_v, out_hbm.at[c, sl])


@functools.cache
def _build_deg():
    return pl.kernel(
        _deg_body,
        out_type=jax.ShapeDtypeStruct((NC, NPAD), jnp.float32),
        mesh=_mesh(),
        scratch_types=[
            pltpu.VMEM((CH, CE), jnp.int32),
            pltpu.VMEM((CH, CE), jnp.int32),
            pltpu.VMEM((CH, CE), jnp.int32),
            pltpu.VMEM((CE,), jnp.float32),
            pltpu.VMEM((RPT,), jnp.float32),
            pltpu.VMEM_SHARED((NPAD,), jnp.float32),
        ],
    )


@functools.cache
def _build_prop():
    return pl.kernel(
        _prop_body,
        out_type=jax.ShapeDtypeStruct((NC, NPAD, D), jnp.float32),
        mesh=_mesh(),
        scratch_types=[
            pltpu.VMEM((CH, CE), jnp.int32),
            pltpu.VMEM((CH, CE), jnp.int32),
            pltpu.VMEM((CH, CE), jnp.int32),
            pltpu.VMEM((CE, D), jnp.float32),
            pltpu.VMEM_SHARED((NPAD, D), jnp.float32),
            pltpu.SemaphoreType.DMA,
        ],
    )


_R = 512


def _dis_from(degT):
    deg = jnp.sum(degT, axis=1, keepdims=True)
    return jnp.where(deg > 0, lax.rsqrt(deg), 0.0)


def _tca_body(degT_ref, x_ref, w0_ref, w1_ref, g_ref, xw0_ref):
    dis = _dis_from(degT_ref[...])
    xb = x_ref[...]
    g_ref[...] = dis * jnp.dot(xb, w1_ref[...],
                               preferred_element_type=jnp.float32)
    xw0_ref[...] = jnp.dot(xb, w0_ref[...], preferred_element_type=jnp.float32)


def _tcb_body(degT_ref, s0_ref, s1_ref, xw0_ref, b_ref, w0_ref, w1_ref,
              g_ref, hw0_ref):
    dis = _dis_from(degT_ref[...])
    h = xw0_ref[...] - dis * (s0_ref[...] + s1_ref[...]) + b_ref[...]
    h = jnp.maximum(h, 0.0)
    g_ref[...] = dis * jnp.dot(h, w1_ref[...],
                               preferred_element_type=jnp.float32)
    hw0_ref[...] = jnp.dot(h, w0_ref[...], preferred_element_type=jnp.float32)


def _tcc_body(degT_ref, s0_ref, s1_ref, hw0_ref, b_ref, out_ref):
    dis = _dis_from(degT_ref[...])
    o = hw0_ref[...] - dis * (s0_ref[...] + s1_ref[...]) + b_ref[...]
    m = jnp.max(o, axis=1, keepdims=True)
    lse = jnp.log(jnp.sum(jnp.exp(o - m), axis=1, keepdims=True)) + m
    out_ref[...] = o - lse


def _row_spec(width):
    return pl.BlockSpec((_R, width), lambda i: (i, 0))


def _full_spec(shape):
    return pl.BlockSpec(shape, lambda i: (0,) * len(shape))


_f32 = jnp.float32


def _tca(degT, x, w0, w1):
    return pl.pallas_call(
        _tca_body,
        grid=(NPAD // _R,),
        in_specs=[_row_spec(NC), _row_spec(D), _full_spec((D, D)),
                  _full_spec((D, D))],
        out_specs=[_row_spec(D), _row_spec(D)],
        out_shape=[jax.ShapeDtypeStruct((NPAD, D), _f32)] * 2,
    )(degT, x, w0, w1)


def _tcb(degT, s0, s1, xw0, b, w0, w1):
    return pl.pallas_call(
        _tcb_body,
        grid=(NPAD // _R,),
        in_specs=[_row_spec(NC), _row_spec(D), _row_spec(D), _row_spec(D),
                  _full_spec((1, D)), _full_spec((D, D)), _full_spec((D, D))],
        out_specs=[_row_spec(D), _row_spec(D)],
        out_shape=[jax.ShapeDtypeStruct((NPAD, D), _f32)] * 2,
    )(degT, s0, s1, xw0, b, w0, w1)


def _tcc(degT, s0, s1, hw0, b):
    return pl.pallas_call(
        _tcc_body,
        grid=(NPAD // _R,),
        in_specs=[_row_spec(NC), _row_spec(D), _row_spec(D), _row_spec(D),
                  _full_spec((1, D))],
        out_specs=_row_spec(D),
        out_shape=jax.ShapeDtypeStruct((NPAD, D), _f32),
    )(degT, s0, s1, hw0, b)


def kernel(x, edge_index, W1, b1, W2, b2):
    row = edge_index[0]
    col = edge_index[1]
    pad = EPAD - E
    zpad = jnp.zeros((pad,), jnp.int32)
    row3 = jnp.concatenate([row, zpad]).reshape(NW, CH, CE)
    col3 = jnp.concatenate([col, zpad]).reshape(NW, CH, CE)
    x_p = jnp.pad(x, ((0, NPAD - N), (0, 0)))
    b1r = b1.reshape(1, D)
    b2r = b2.reshape(1, D)

    deg_call = _build_deg()
    prop_call = _build_prop()
    deg_parts = deg_call(row3, col3)
    degT = deg_parts.T
    g1, xw0 = _tca(degT, x_p, W1[0], W1[1])
    s1 = prop_call(g1, row3, col3)
    g2, hw0 = _tcb(degT, s1[0], s1[1], xw0, b1r, W2[0], W2[1])
    s2 = prop_call(g2, row3, col3)
    out = _tcc(degT, s2[0], s2[1], hw0, b2r)
    return out[:N]


# speedup vs baseline: 9.5593x; 1.3965x over previous
import functools

import jax
import jax.numpy as jnp
from jax import lax
from jax.experimental import pallas as pl
from jax.experimental.pallas import tpu as pltpu
from jax.experimental.pallas import tpu_sc as plsc

N = 10000
D = 128
DH = D // 2
E = 320000
NC = 2
NS = 16
CE = 128
CH = 160
CHH = CH // NC
EPT = CH * CE
EPAD = NS * EPT
NPAD = 10240
RPT = NPAD // NS
TRASH = N


def _mesh():
    return plsc.VectorSubcoreMesh(core_axis_name="c", subcore_axis_name="s")


def _deg_body(row_hbm, col_hbm, out_hbm, ridx_v, col_v, ones_v, buf_v,
              hist_sh):
    c = lax.axis_index("c")
    s = lax.axis_index("s")
    base = c * CHH
    pltpu.sync_copy(row_hbm.at[s, pl.ds(base, CHH)], ridx_v)
    pltpu.sync_copy(col_hbm.at[s, pl.ds(base, CHH)], col_v)
    one = jnp.ones((16,), jnp.float32)
    zero = jnp.zeros((16,), jnp.float32)
    for k in range(CE // 16):
        ones_v[pl.ds(k * 16, 16)] = one

    def _remap(j, carry):
        for k in range(CE // 16):
            sl = pl.ds(k * 16, 16)
            r = ridx_v[j, sl]
            cc = col_v[j, sl]
            ridx_v[j, sl] = jnp.where(r == cc, TRASH, r)
        return carry

    lax.fori_loop(0, CHH, _remap, 0)

    def _zb(i, carry):
        buf_v[pl.ds(i * 16, 16)] = zero
        return carry

    lax.fori_loop(0, RPT // 16, _zb, 0)
    pltpu.sync_copy(buf_v, hist_sh.at[pl.ds(s * RPT, RPT)])
    plsc.subcore_barrier()

    def _acc(j, carry):
        pltpu.sync_copy(ones_v, hist_sh.at[ridx_v.at[j]], add=True)
        return carry

    lax.fori_loop(0, CHH, _acc, 0)
    plsc.subcore_barrier()
    pltpu.sync_copy(hist_sh.at[pl.ds(s * RPT, RPT)], buf_v)
    pltpu.sync_copy(buf_v, out_hbm.at[c, pl.ds(s * RPT, RPT)])


def _prop_body(g_hbm, row_hbm, col_hbm, out_hbm, row_v, cfx_v, rows0_v,
               rows1_v, acc_sh, gsem0, gsem1, ssem0, ssem1):
    c = lax.axis_index("c")
    s = lax.axis_index("s")
    pltpu.sync_copy(row_hbm.at[s], row_v)
    pltpu.sync_copy(col_hbm.at[s], cfx_v)

    def _remap(j, carry):
        for k in range(CE // 16):
            sl = pl.ds(k * 16, 16)
            r = row_v[j, sl]
            cc = cfx_v[j, sl]
            cfx_v[j, sl] = jnp.where(r == cc, TRASH, cc)
        return carry

    lax.fori_loop(0, CH, _remap, 0)

    zero = jnp.zeros((16,), jnp.float32)

    def _zb(i, carry):
        for k in range(DH // 16):
            rows0_v[i, pl.ds(k * 16, 16)] = zero
        return carry

    lax.fori_loop(0, CE, _zb, 0)
    for t in range(RPT // CE):
        pltpu.sync_copy(rows0_v, acc_sh.at[pl.ds(s * RPT + t * CE, CE)])
    plsc.subcore_barrier()

    gh = g_hbm.at[c]

    bufs = (rows0_v, gsem0, ssem0)
    nbufs = (rows1_v, gsem1, ssem1)
    pltpu.async_copy(gh.at[row_v.at[0]], rows0_v, gsem0)
    for j in range(CH):
        buf, gsem, ssem = bufs if j % 2 == 0 else nbufs
        nbuf, ngsem, nssem = nbufs if j % 2 == 0 else bufs
        pltpu.make_async_copy(gh.at[row_v.at[j]], buf, gsem).wait()
        if j + 1 < CH:
            if j >= 1:
                pltpu.make_async_copy(nbuf, acc_sh.at[cfx_v.at[j - 1]],
                                      nssem).wait()
            pltpu.async_copy(gh.at[row_v.at[j + 1]], nbuf, ngsem)
        pltpu.async_copy(buf, acc_sh.at[cfx_v.at[j]], ssem, add=True)
    last, _, lssem = bufs if (CH - 1) % 2 == 0 else nbufs
    prev, _, pssem = nbufs if (CH - 1) % 2 == 0 else bufs
    pltpu.make_async_copy(prev, acc_sh.at[cfx_v.at[CH - 2]], pssem).wait()
    pltpu.make_async_copy(last, acc_sh.at[cfx_v.at[CH - 1]], lssem).wait()
    plsc.subcore_barrier()
    for t in range(RPT // CE):
        r0 = s * RPT + t * CE
        pltpu.sync_copy(acc_sh.at[pl.ds(r0, CE)], rows0_v)
        pltpu.sync_copy(rows0_v, out_hbm.at[c, pl.ds(r0, CE)])


@functools.cache
def _build_deg():
    return pl.kernel(
        _deg_body,
        out_type=jax.ShapeDtypeStruct((NC, NPAD), jnp.float32),
        mesh=_mesh(),
        scratch_types=[
            pltpu.VMEM((CHH, CE), jnp.int32),
            pltpu.VMEM((CHH, CE), jnp.int32),
            pltpu.VMEM((CE,), jnp.float32),
            pltpu.VMEM((RPT,), jnp.float32),
            pltpu.VMEM_SHARED((NPAD,), jnp.float32),
        ],
    )


@functools.cache
def _build_prop():
    return pl.kernel(
        _prop_body,
        out_type=jax.ShapeDtypeStruct((NC, NPAD, DH), jnp.float32),
        mesh=_mesh(),
        compiler_params=pltpu.CompilerParams(use_tc_tiling_on_sc=False),
        scratch_types=[
            pltpu.VMEM((CH, CE), jnp.int32),
            pltpu.VMEM((CH, CE), jnp.int32),
            pltpu.VMEM((CE, DH), jnp.float32),
            pltpu.VMEM((CE, DH), jnp.float32),
            pltpu.VMEM_SHARED((NPAD, DH), jnp.float32),
            pltpu.SemaphoreType.DMA,
            pltpu.SemaphoreType.DMA,
            pltpu.SemaphoreType.DMA,
            pltpu.SemaphoreType.DMA,
        ],
    )


_R = 512


def _dis_from(degT):
    deg = jnp.sum(degT, axis=1, keepdims=True)
    return jnp.where(deg > 0, lax.rsqrt(deg), 0.0)


def _tca_body(degT_ref, x_ref, w0_ref, w1_ref, g_ref, xw0_ref):
    dis = _dis_from(degT_ref[...])
    xb = x_ref[...]
    g = dis * jnp.dot(xb, w1_ref[...], preferred_element_type=jnp.float32)
    g_ref[0] = g[:, :DH]
    g_ref[1] = g[:, DH:]
    xw0_ref[...] = jnp.dot(xb, w0_ref[...], preferred_element_type=jnp.float32)


def _tcb_body(degT_ref, s_ref, xw0_ref, b_ref, w0_ref, w1_ref,
              g_ref, hw0_ref):
    dis = _dis_from(degT_ref[...])
    sfull = jnp.concatenate([s_ref[0], s_ref[1]], axis=1)
    h = xw0_ref[...] - dis * sfull + b_ref[...]
    h = jnp.maximum(h, 0.0)
    g = dis * jnp.dot(h, w1_ref[...], preferred_element_type=jnp.float32)
    g_ref[0] = g[:, :DH]
    g_ref[1] = g[:, DH:]
    hw0_ref[...] = jnp.dot(h, w0_ref[...], preferred_element_type=jnp.float32)


def _tcc_body(degT_ref, s_ref, hw0_ref, b_ref, out_ref):
    dis = _dis_from(degT_ref[...])
    sfull = jnp.concatenate([s_ref[0], s_ref[1]], axis=1)
    o = hw0_ref[...] - dis * sfull + b_ref[...]
    m = jnp.max(o, axis=1, keepdims=True)
    lse = jnp.log(jnp.sum(jnp.exp(o - m), axis=1, keepdims=True)) + m
    out_ref[...] = o - lse


def _row_spec(width):
    return pl.BlockSpec((_R, width), lambda i: (i, 0))


def _g_spec():
    return pl.BlockSpec((NC, _R, DH), lambda i: (0, i, 0))


def _full_spec(shape):
    return pl.BlockSpec(shape, lambda i: (0,) * len(shape))


_f32 = jnp.float32


def _tca(degT, x, w0, w1):
    return pl.pallas_call(
        _tca_body,
        grid=(NPAD // _R,),
        in_specs=[_row_spec(NC), _row_spec(D), _full_spec((D, D)),
                  _full_spec((D, D))],
        out_specs=[_g_spec(), _row_spec(D)],
        out_shape=[jax.ShapeDtypeStruct((NC, NPAD, DH), _f32),
                   jax.ShapeDtypeStruct((NPAD, D), _f32)],
    )(degT, x, w0, w1)


def _tcb(degT, s, xw0, b, w0, w1):
    return pl.pallas_call(
        _tcb_body,
        grid=(NPAD // _R,),
        in_specs=[_row_spec(NC), _g_spec(), _row_spec(D),
                  _full_spec((1, D)), _full_spec((D, D)), _full_spec((D, D))],
        out_specs=[_g_spec(), _row_spec(D)],
        out_shape=[jax.ShapeDtypeStruct((NC, NPAD, DH), _f32),
                   jax.ShapeDtypeStruct((NPAD, D), _f32)],
    )(degT, s, xw0, b, w0, w1)


def _tcc(degT, s, hw0, b):
    return pl.pallas_call(
        _tcc_body,
        grid=(NPAD // _R,),
        in_specs=[_row_spec(NC), _g_spec(), _row_spec(D),
                  _full_spec((1, D))],
        out_specs=_row_spec(D),
        out_shape=jax.ShapeDtypeStruct((NPAD, D), _f32),
    )(degT, s, hw0, b)


def kernel(x, edge_index, W1, b1, W2, b2):
    row = edge_index[0]
    col = edge_index[1]
    pad = EPAD - E
    zpad = jnp.zeros((pad,), jnp.int32)
    row3 = jnp.concatenate([row, zpad]).reshape(NS, CH, CE)
    col3 = jnp.concatenate([col, zpad]).reshape(NS, CH, CE)
    x_p = jnp.pad(x, ((0, NPAD - N), (0, 0)))
    b1r = b1.reshape(1, D)
    b2r = b2.reshape(1, D)

    deg_call = _build_deg()
    prop_call = _build_prop()
    deg_parts = deg_call(row3, col3)
    degT = deg_parts.T
    g1, xw0 = _tca(degT, x_p, W1[0], W1[1])
    s1 = prop_call(g1, row3, col3)
    g2, hw0 = _tcb(degT, s1, xw0, b1r, W2[0], W2[1])
    s2 = prop_call(g2, row3, col3)
    out = _tcc(degT, s2, hw0, b2r)
    return out[:N]


# speedup vs baseline: 10.8018x; 1.1300x over previous
import functools

import jax
import jax.numpy as jnp
from jax import lax
from jax.experimental import pallas as pl
from jax.experimental.pallas import tpu as pltpu
from jax.experimental.pallas import tpu_sc as plsc

N = 10000
D = 128
DH = D // 2
E = 320000
NC = 2
NS = 16
CE = 128
CH = 160
CHH = CH // NC
EPT = CH * CE
EPAD = NS * EPT
NPAD = 10240
RPT = NPAD // NS
TRASH = N


def _mesh():
    return plsc.VectorSubcoreMesh(core_axis_name="c", subcore_axis_name="s")


def _deg_body(row_hbm, col_hbm, out_hbm, ridx_v, col_v, ones_v, buf_v,
              hist_sh):
    c = lax.axis_index("c")
    s = lax.axis_index("s")
    base = c * CHH
    pltpu.sync_copy(row_hbm.at[s, pl.ds(base, CHH)], ridx_v)
    pltpu.sync_copy(col_hbm.at[s, pl.ds(base, CHH)], col_v)
    one = jnp.ones((16,), jnp.float32)
    zero = jnp.zeros((16,), jnp.float32)
    for k in range(CE // 16):
        ones_v[pl.ds(k * 16, 16)] = one

    def _remap(j, carry):
        for k in range(CE // 16):
            sl = pl.ds(k * 16, 16)
            r = ridx_v[j, sl]
            cc = col_v[j, sl]
            ridx_v[j, sl] = jnp.where(r == cc, TRASH, r)
        return carry

    lax.fori_loop(0, CHH, _remap, 0)

    def _zb(i, carry):
        buf_v[pl.ds(i * 16, 16)] = zero
        return carry

    lax.fori_loop(0, RPT // 16, _zb, 0)
    pltpu.sync_copy(buf_v, hist_sh.at[pl.ds(s * RPT, RPT)])
    plsc.subcore_barrier()

    def _acc(j, carry):
        pltpu.sync_copy(ones_v, hist_sh.at[ridx_v.at[j]], add=True)
        return carry

    lax.fori_loop(0, CHH, _acc, 0)
    plsc.subcore_barrier()
    pltpu.sync_copy(hist_sh.at[pl.ds(s * RPT, RPT)], buf_v)
    pltpu.sync_copy(buf_v, out_hbm.at[c, pl.ds(s * RPT, RPT)])


def _prop_body(g_hbm, row_hbm, col_hbm, out_hbm, row_v, cfx_v, rows0_v,
               rows1_v, rows2_v, rows3_v, acc_sh, gsem0, gsem1, gsem2, gsem3,
               ssem0, ssem1, ssem2, ssem3):
    c = lax.axis_index("c")
    s = lax.axis_index("s")
    pltpu.sync_copy(row_hbm.at[s], row_v)
    pltpu.sync_copy(col_hbm.at[s], cfx_v)

    def _remap(j, carry):
        for k in range(CE // 16):
            sl = pl.ds(k * 16, 16)
            r = row_v[j, sl]
            cc = cfx_v[j, sl]
            cfx_v[j, sl] = jnp.where(r == cc, TRASH, cc)
        return carry

    lax.fori_loop(0, CH, _remap, 0)

    zero = jnp.zeros((16,), jnp.float32)

    def _zb(i, carry):
        for k in range(DH // 16):
            rows0_v[i, pl.ds(k * 16, 16)] = zero
        return carry

    lax.fori_loop(0, CE, _zb, 0)
    for t in range(RPT // CE):
        pltpu.sync_copy(rows0_v, acc_sh.at[pl.ds(s * RPT + t * CE, CE)])
    plsc.subcore_barrier()

    gh = g_hbm.at[c]

    bufs = (rows0_v, rows1_v, rows2_v, rows3_v)
    gsems = (gsem0, gsem1, gsem2, gsem3)
    ssems = (ssem0, ssem1, ssem2, ssem3)
    NBUF = 4
    pltpu.async_copy(gh.at[row_v.at[0]], bufs[0], gsems[0])
    pltpu.async_copy(gh.at[row_v.at[1]], bufs[1], gsems[1])
    for j in range(CH):
        b = j % NBUF
        pltpu.make_async_copy(gh.at[row_v.at[j]], bufs[b], gsems[b]).wait()
        pltpu.async_copy(bufs[b], acc_sh.at[cfx_v.at[j]], ssems[b], add=True)
        jn = j + 2
        if jn < CH:
            bn = jn % NBUF
            if j >= 2:
                pltpu.make_async_copy(bufs[bn], acc_sh.at[cfx_v.at[j - 2]],
                                      ssems[bn]).wait()
            pltpu.async_copy(gh.at[row_v.at[jn]], bufs[bn], gsems[bn])
    for jj in range(CH - 4, CH):
        pltpu.make_async_copy(bufs[jj % NBUF], acc_sh.at[cfx_v.at[jj]],
                              ssems[jj % NBUF]).wait()
    plsc.subcore_barrier()
    for t in range(RPT // CE):
        r0 = s * RPT + t * CE
        pltpu.sync_copy(acc_sh.at[pl.ds(r0, CE)], rows0_v)
        pltpu.sync_copy(rows0_v, out_hbm.at[c, pl.ds(r0, CE)])


@functools.cache
def _build_deg():
    return pl.kernel(
        _deg_body,
        out_type=jax.ShapeDtypeStruct((NC, NPAD), jnp.float32),
        mesh=_mesh(),
        scratch_types=[
            pltpu.VMEM((CHH, CE), jnp.int32),
            pltpu.VMEM((CHH, CE), jnp.int32),
            pltpu.VMEM((CE,), jnp.float32),
            pltpu.VMEM((RPT,), jnp.float32),
            pltpu.VMEM_SHARED((NPAD,), jnp.float32),
        ],
    )


@functools.cache
def _build_prop():
    return pl.kernel(
        _prop_body,
        out_type=jax.ShapeDtypeStruct((NC, NPAD, DH), jnp.float32),
        mesh=_mesh(),
        compiler_params=pltpu.CompilerParams(use_tc_tiling_on_sc=False),
        scratch_types=[
            pltpu.VMEM((CH, CE), jnp.int32),
            pltpu.VMEM((CH, CE), jnp.int32),
            pltpu.VMEM((CE, DH), jnp.float32),
            pltpu.VMEM((CE, DH), jnp.float32),
            pltpu.VMEM((CE, DH), jnp.float32),
            pltpu.VMEM((CE, DH), jnp.float32),
            pltpu.VMEM_SHARED((NPAD, DH), jnp.float32),
        ] + [pltpu.SemaphoreType.DMA] * 8,
    )


_R = 512


def _dis_from(degT):
    deg = jnp.sum(degT, axis=1, keepdims=True)
    return jnp.where(deg > 0, lax.rsqrt(deg), 0.0)


def _tca_body(degT_ref, x_ref, w0_ref, w1_ref, g_ref, xw0_ref):
    dis = _dis_from(degT_ref[...])
    xb = x_ref[...]
    g = dis * jnp.dot(xb, w1_ref[...], preferred_element_type=jnp.float32)
    g_ref[0] = g[:, :DH]
    g_ref[1] = g[:, DH:]
    xw0_ref[...] = jnp.dot(xb, w0_ref[...], preferred_element_type=jnp.float32)


def _tcb_body(degT_ref, s_ref, xw0_ref, b_ref, w0_ref, w1_ref,
              g_ref, hw0_ref):
    dis = _dis_from(degT_ref[...])
    sfull = jnp.concatenate([s_ref[0], s_ref[1]], axis=1)
    h = xw0_ref[...] - dis * sfull + b_ref[...]
    h = jnp.maximum(h, 0.0)
    g = dis * jnp.dot(h, w1_ref[...], preferred_element_type=jnp.float32)
    g_ref[0] = g[:, :DH]
    g_ref[1] = g[:, DH:]
    hw0_ref[...] = jnp.dot(h, w0_ref[...], preferred_element_type=jnp.float32)


def _tcc_body(degT_ref, s_ref, hw0_ref, b_ref, out_ref):
    dis = _dis_from(degT_ref[...])
    sfull = jnp.concatenate([s_ref[0], s_ref[1]], axis=1)
    o = hw0_ref[...] - dis * sfull + b_ref[...]
    m = jnp.max(o, axis=1, keepdims=True)
    lse = jnp.log(jnp.sum(jnp.exp(o - m), axis=1, keepdims=True)) + m
    out_ref[...] = o - lse


def _row_spec(width):
    return pl.BlockSpec((_R, width), lambda i: (i, 0))


def _g_spec():
    return pl.BlockSpec((NC, _R, DH), lambda i: (0, i, 0))


def _full_spec(shape):
    return pl.BlockSpec(shape, lambda i: (0,) * len(shape))


_f32 = jnp.float32


def _tca(degT, x, w0, w1):
    return pl.pallas_call(
        _tca_body,
        grid=(NPAD // _R,),
        in_specs=[_row_spec(NC), _row_spec(D), _full_spec((D, D)),
                  _full_spec((D, D))],
        out_specs=[_g_spec(), _row_spec(D)],
        out_shape=[jax.ShapeDtypeStruct((NC, NPAD, DH), _f32),
                   jax.ShapeDtypeStruct((NPAD, D), _f32)],
    )(degT, x, w0, w1)


def _tcb(degT, s, xw0, b, w0, w1):
    return pl.pallas_call(
        _tcb_body,
        grid=(NPAD // _R,),
        in_specs=[_row_spec(NC), _g_spec(), _row_spec(D),
                  _full_spec((1, D)), _full_spec((D, D)), _full_spec((D, D))],
        out_specs=[_g_spec(), _row_spec(D)],
        out_shape=[jax.ShapeDtypeStruct((NC, NPAD, DH), _f32),
                   jax.ShapeDtypeStruct((NPAD, D), _f32)],
    )(degT, s, xw0, b, w0, w1)


def _tcc(degT, s, hw0, b):
    return pl.pallas_call(
        _tcc_body,
        grid=(NPAD // _R,),
        in_specs=[_row_spec(NC), _g_spec(), _row_spec(D),
                  _full_spec((1, D))],
        out_specs=_row_spec(D),
        out_shape=jax.ShapeDtypeStruct((NPAD, D), _f32),
    )(degT, s, hw0, b)


def kernel(x, edge_index, W1, b1, W2, b2):
    row = edge_index[0]
    col = edge_index[1]
    pad = EPAD - E
    zpad = jnp.zeros((pad,), jnp.int32)
    row3 = jnp.concatenate([row, zpad]).reshape(NS, CH, CE)
    col3 = jnp.concatenate([col, zpad]).reshape(NS, CH, CE)
    x_p = jnp.pad(x, ((0, NPAD - N), (0, 0)))
    b1r = b1.reshape(1, D)
    b2r = b2.reshape(1, D)

    deg_call = _build_deg()
    prop_call = _build_prop()
    deg_parts = deg_call(row3, col3)
    degT = deg_parts.T
    g1, xw0 = _tca(degT, x_p, W1[0], W1[1])
    s1 = prop_call(g1, row3, col3)
    g2, hw0 = _tcb(degT, s1, xw0, b1r, W2[0], W2[1])
    s2 = prop_call(g2, row3, col3)
    out = _tcc(degT, s2, hw0, b2r)
    return out[:N]


# speedup vs baseline: 10.9065x; 1.0097x over previous
import functools

import jax
import jax.numpy as jnp
from jax import lax
from jax.experimental import pallas as pl
from jax.experimental.pallas import tpu as pltpu
from jax.experimental.pallas import tpu_sc as plsc

N = 10000
D = 128
DH = D // 2
E = 320000
NC = 2
NS = 16
CE = 128
CH = 160
CHH = CH // NC
EPT = CH * CE
EPAD = NS * EPT
NPAD = 10240
RPT = NPAD // NS
TRASH = N


def _mesh():
    return plsc.VectorSubcoreMesh(core_axis_name="c", subcore_axis_name="s")


def _deg_body(row_hbm, col_hbm, out_hbm, ridx_v, col_v, ones_v, buf_v,
              hist_sh):
    c = lax.axis_index("c")
    s = lax.axis_index("s")
    base = c * CHH
    pltpu.sync_copy(row_hbm.at[s, pl.ds(base, CHH)], ridx_v)
    pltpu.sync_copy(col_hbm.at[s, pl.ds(base, CHH)], col_v)
    one = jnp.ones((16,), jnp.float32)
    zero = jnp.zeros((16,), jnp.float32)
    for k in range(CE // 16):
        ones_v[pl.ds(k * 16, 16)] = one

    def _remap(j, carry):
        for k in range(CE // 16):
            sl = pl.ds(k * 16, 16)
            r = ridx_v[j, sl]
            cc = col_v[j, sl]
            ridx_v[j, sl] = jnp.where(r == cc, TRASH, r)
        return carry

    lax.fori_loop(0, CHH, _remap, 0)

    def _zb(i, carry):
        buf_v[pl.ds(i * 16, 16)] = zero
        return carry

    lax.fori_loop(0, RPT // 16, _zb, 0)
    pltpu.sync_copy(buf_v, hist_sh.at[pl.ds(s * RPT, RPT)])
    plsc.subcore_barrier()

    def _acc(j, carry):
        pltpu.sync_copy(ones_v, hist_sh.at[ridx_v.at[j]], add=True)
        return carry

    lax.fori_loop(0, CHH, _acc, 0)
    plsc.subcore_barrier()
    pltpu.sync_copy(hist_sh.at[pl.ds(s * RPT, RPT)], buf_v)
    pltpu.sync_copy(buf_v, out_hbm.at[c, pl.ds(s * RPT, RPT)])


NBUF = 6


def _prop_body(g_hbm, row_hbm, col_hbm, out_hbm, row_v, cfx_v, *scr):
    bufs = scr[:NBUF]
    acc_sh = scr[NBUF]
    gsems = scr[NBUF + 1:2 * NBUF + 1]
    ssems = scr[2 * NBUF + 1:3 * NBUF + 1]
    rows0_v = bufs[0]
    c = lax.axis_index("c")
    s = lax.axis_index("s")
    pltpu.sync_copy(row_hbm.at[s], row_v)
    pltpu.sync_copy(col_hbm.at[s], cfx_v)

    def _remap(j, carry):
        for k in range(CE // 16):
            sl = pl.ds(k * 16, 16)
            r = row_v[j, sl]
            cc = cfx_v[j, sl]
            cfx_v[j, sl] = jnp.where(r == cc, TRASH, cc)
        return carry

    lax.fori_loop(0, CH, _remap, 0)

    zero = jnp.zeros((16,), jnp.float32)

    def _zb(i, carry):
        for k in range(DH // 16):
            rows0_v[i, pl.ds(k * 16, 16)] = zero
        return carry

    lax.fori_loop(0, CE, _zb, 0)
    for t in range(RPT // CE):
        pltpu.sync_copy(rows0_v, acc_sh.at[pl.ds(s * RPT + t * CE, CE)])
    plsc.subcore_barrier()

    gh = g_hbm.at[c]

    A = NBUF // 2
    for a in range(A):
        pltpu.async_copy(gh.at[row_v.at[a]], bufs[a], gsems[a])
    for j in range(CH):
        b = j % NBUF
        pltpu.make_async_copy(gh.at[row_v.at[j]], bufs[b], gsems[b]).wait()
        pltpu.async_copy(bufs[b], acc_sh.at[cfx_v.at[j]], ssems[b], add=True)
        jn = j + A
        if jn < CH:
            bn = jn % NBUF
            jp = jn - NBUF
            if jp >= 0:
                pltpu.make_async_copy(bufs[bn], acc_sh.at[cfx_v.at[jp]],
                                      ssems[bn]).wait()
            pltpu.async_copy(gh.at[row_v.at[jn]], bufs[bn], gsems[bn])
    for jj in range(max(CH - NBUF, 0), CH):
        pltpu.make_async_copy(bufs[jj % NBUF], acc_sh.at[cfx_v.at[jj]],
                              ssems[jj % NBUF]).wait()
    plsc.subcore_barrier()
    for t in range(RPT // CE):
        r0 = s * RPT + t * CE
        pltpu.sync_copy(acc_sh.at[pl.ds(r0, CE)], rows0_v)
        pltpu.sync_copy(rows0_v, out_hbm.at[c, pl.ds(r0, CE)])


@functools.cache
def _build_deg():
    return pl.kernel(
        _deg_body,
        out_type=jax.ShapeDtypeStruct((NC, NPAD), jnp.float32),
        mesh=_mesh(),
        scratch_types=[
            pltpu.VMEM((CHH, CE), jnp.int32),
            pltpu.VMEM((CHH, CE), jnp.int32),
            pltpu.VMEM((CE,), jnp.float32),
            pltpu.VMEM((RPT,), jnp.float32),
            pltpu.VMEM_SHARED((NPAD,), jnp.float32),
        ],
    )


@functools.cache
def _build_prop():
    return pl.kernel(
        _prop_body,
        out_type=jax.ShapeDtypeStruct((NC, NPAD, DH), jnp.float32),
        mesh=_mesh(),
        compiler_params=pltpu.CompilerParams(use_tc_tiling_on_sc=False),
        scratch_types=[
            pltpu.VMEM((CH, CE), jnp.int32),
            pltpu.VMEM((CH, CE), jnp.int32),
        ] + [pltpu.VMEM((CE, DH), jnp.float32)] * NBUF + [
            pltpu.VMEM_SHARED((NPAD, DH), jnp.float32),
        ] + [pltpu.SemaphoreType.DMA] * (2 * NBUF),
    )


_R = 512


def _dis_from(degT):
    deg = jnp.sum(degT, axis=1, keepdims=True)
    return jnp.where(deg > 0, lax.rsqrt(deg), 0.0)


def _tca_body(degT_ref, x_ref, w0_ref, w1_ref, g_ref, xw0_ref):
    dis = _dis_from(degT_ref[...])
    xb = x_ref[...]
    g = dis * jnp.dot(xb, w1_ref[...], preferred_element_type=jnp.float32)
    g_ref[0] = g[:, :DH]
    g_ref[1] = g[:, DH:]
    xw0_ref[...] = jnp.dot(xb, w0_ref[...], preferred_element_type=jnp.float32)


def _tcb_body(degT_ref, s_ref, xw0_ref, b_ref, w0_ref, w1_ref,
              g_ref, hw0_ref):
    dis = _dis_from(degT_ref[...])
    sfull = jnp.concatenate([s_ref[0], s_ref[1]], axis=1)
    h = xw0_ref[...] - dis * sfull + b_ref[...]
    h = jnp.maximum(h, 0.0)
    g = dis * jnp.dot(h, w1_ref[...], preferred_element_type=jnp.float32)
    g_ref[0] = g[:, :DH]
    g_ref[1] = g[:, DH:]
    hw0_ref[...] = jnp.dot(h, w0_ref[...], preferred_element_type=jnp.float32)


def _tcc_body(degT_ref, s_ref, hw0_ref, b_ref, out_ref):
    dis = _dis_from(degT_ref[...])
    sfull = jnp.concatenate([s_ref[0], s_ref[1]], axis=1)
    o = hw0_ref[...] - dis * sfull + b_ref[...]
    m = jnp.max(o, axis=1, keepdims=True)
    lse = jnp.log(jnp.sum(jnp.exp(o - m), axis=1, keepdims=True)) + m
    out_ref[...] = o - lse


def _row_spec(width):
    return pl.BlockSpec((_R, width), lambda i: (i, 0))


def _g_spec():
    return pl.BlockSpec((NC, _R, DH), lambda i: (0, i, 0))


def _full_spec(shape):
    return pl.BlockSpec(shape, lambda i: (0,) * len(shape))


_f32 = jnp.float32


def _tca(degT, x, w0, w1):
    return pl.pallas_call(
        _tca_body,
        grid=(NPAD // _R,),
        in_specs=[_row_spec(NC), _row_spec(D), _full_spec((D, D)),
                  _full_spec((D, D))],
        out_specs=[_g_spec(), _row_spec(D)],
        out_shape=[jax.ShapeDtypeStruct((NC, NPAD, DH), _f32),
                   jax.ShapeDtypeStruct((NPAD, D), _f32)],
    )(degT, x, w0, w1)


def _tcb(degT, s, xw0, b, w0, w1):
    return pl.pallas_call(
        _tcb_body,
        grid=(NPAD // _R,),
        in_specs=[_row_spec(NC), _g_spec(), _row_spec(D),
                  _full_spec((1, D)), _full_spec((D, D)), _full_spec((D, D))],
        out_specs=[_g_spec(), _row_spec(D)],
        out_shape=[jax.ShapeDtypeStruct((NC, NPAD, DH), _f32),
                   jax.ShapeDtypeStruct((NPAD, D), _f32)],
    )(degT, s, xw0, b, w0, w1)


def _tcc(degT, s, hw0, b):
    return pl.pallas_call(
        _tcc_body,
        grid=(NPAD // _R,),
        in_specs=[_row_spec(NC), _g_spec(), _row_spec(D),
                  _full_spec((1, D))],
        out_specs=_row_spec(D),
        out_shape=jax.ShapeDtypeStruct((NPAD, D), _f32),
    )(degT, s, hw0, b)


def kernel(x, edge_index, W1, b1, W2, b2):
    row = edge_index[0]
    col = edge_index[1]
    pad = EPAD - E
    zpad = jnp.zeros((pad,), jnp.int32)
    row3 = jnp.concatenate([row, zpad]).reshape(NS, CH, CE)
    col3 = jnp.concatenate([col, zpad]).reshape(NS, CH, CE)
    x_p = jnp.pad(x, ((0, NPAD - N), (0, 0)))
    b1r = b1.reshape(1, D)
    b2r = b2.reshape(1, D)

    deg_call = _build_deg()
    prop_call = _build_prop()
    deg_parts = deg_call(row3, col3)
    degT = deg_parts.T
    g1, xw0 = _tca(degT, x_p, W1[0], W1[1])
    s1 = prop_call(g1, row3, col3)
    g2, hw0 = _tcb(degT, s1, xw0, b1r, W2[0], W2[1])
    s2 = prop_call(g2, row3, col3)
    out = _tcc(degT, s2, hw0, b2r)
    return out[:N]


# speedup vs baseline: 11.1373x; 1.0212x over previous
import functools

import jax
import jax.numpy as jnp
from jax import lax
from jax.experimental import pallas as pl
from jax.experimental.pallas import tpu as pltpu
from jax.experimental.pallas import tpu_sc as plsc

N = 10000
D = 128
DH = D // 2
E = 320000
NC = 2
NS = 16
CE = 128
CH = 160
CHH = CH // NC
EPT = CH * CE
EPAD = NS * EPT
NPAD = 10240
RPT = NPAD // NS
TRASH = N


def _mesh():
    return plsc.VectorSubcoreMesh(core_axis_name="c", subcore_axis_name="s")


def _deg_body(row_hbm, col_hbm, out_hbm, cfx_hbm, ridx_v, col_v, ones_v,
              buf_v, hist_sh):
    c = lax.axis_index("c")
    s = lax.axis_index("s")
    base = c * CHH
    pltpu.sync_copy(row_hbm.at[s, pl.ds(base, CHH)], ridx_v)
    pltpu.sync_copy(col_hbm.at[s, pl.ds(base, CHH)], col_v)
    one = jnp.ones((16,), jnp.float32)
    zero = jnp.zeros((16,), jnp.float32)
    for k in range(CE // 16):
        ones_v[pl.ds(k * 16, 16)] = one

    def _remap(j, carry):
        for k in range(CE // 16):
            sl = pl.ds(k * 16, 16)
            r = ridx_v[j, sl]
            cc = col_v[j, sl]
            loop = r == cc
            ridx_v[j, sl] = jnp.where(loop, TRASH, r)
            col_v[j, sl] = jnp.where(loop, TRASH, cc)
        return carry

    lax.fori_loop(0, CHH, _remap, 0)
    pltpu.sync_copy(col_v, cfx_hbm.at[s, pl.ds(base, CHH)])

    def _zb(i, carry):
        buf_v[pl.ds(i * 16, 16)] = zero
        return carry

    lax.fori_loop(0, RPT // 16, _zb, 0)
    pltpu.sync_copy(buf_v, hist_sh.at[pl.ds(s * RPT, RPT)])
    plsc.subcore_barrier()

    def _acc(j, carry):
        pltpu.sync_copy(ones_v, hist_sh.at[ridx_v.at[j]], add=True)
        return carry

    lax.fori_loop(0, CHH, _acc, 0)
    plsc.subcore_barrier()
    pltpu.sync_copy(hist_sh.at[pl.ds(s * RPT, RPT)], buf_v)
    pltpu.sync_copy(buf_v, out_hbm.at[c, pl.ds(s * RPT, RPT)])


NBUF = 6


def _prop_body(g_hbm, row_hbm, cfx_hbm, out_hbm, row_v, cfx_v, *scr):
    bufs = scr[:NBUF]
    acc_sh = scr[NBUF]
    gsems = scr[NBUF + 1:2 * NBUF + 1]
    ssems = scr[2 * NBUF + 1:3 * NBUF + 1]
    rows0_v = bufs[0]
    c = lax.axis_index("c")
    s = lax.axis_index("s")
    pltpu.sync_copy(row_hbm.at[s], row_v)
    pltpu.sync_copy(cfx_hbm.at[s], cfx_v)

    zero = jnp.zeros((16,), jnp.float32)

    def _zb(i, carry):
        for k in range(DH // 16):
            rows0_v[i, pl.ds(k * 16, 16)] = zero
        return carry

    lax.fori_loop(0, CE, _zb, 0)
    for t in range(RPT // CE):
        pltpu.sync_copy(rows0_v, acc_sh.at[pl.ds(s * RPT + t * CE, CE)])

    gh = g_hbm.at[c]

    A = NBUF // 2
    for a in range(A):
        pltpu.async_copy(gh.at[row_v.at[a]], bufs[a], gsems[a])
    plsc.subcore_barrier()
    for j in range(CH):
        b = j % NBUF
        pltpu.make_async_copy(gh.at[row_v.at[j]], bufs[b], gsems[b]).wait()
        pltpu.async_copy(bufs[b], acc_sh.at[cfx_v.at[j]], ssems[b], add=True)
        jn = j + A
        if jn < CH:
            bn = jn % NBUF
            jp = jn - NBUF
            if jp >= 0:
                pltpu.make_async_copy(bufs[bn], acc_sh.at[cfx_v.at[jp]],
                                      ssems[bn]).wait()
            pltpu.async_copy(gh.at[row_v.at[jn]], bufs[bn], gsems[bn])
    for jj in range(max(CH - NBUF, 0), CH):
        pltpu.make_async_copy(bufs[jj % NBUF], acc_sh.at[cfx_v.at[jj]],
                              ssems[jj % NBUF]).wait()
    plsc.subcore_barrier()
    pltpu.sync_copy(acc_sh.at[pl.ds(s * RPT, RPT)], out_hbm.at[c, pl.ds(s * RPT, RPT)])


@functools.cache
def _build_deg():
    return pl.kernel(
        _deg_body,
        out_type=(jax.ShapeDtypeStruct((NC, NPAD), jnp.float32),
                  jax.ShapeDtypeStruct((NS, CH, CE), jnp.int32)),
        mesh=_mesh(),
        scratch_types=[
            pltpu.VMEM((CHH, CE), jnp.int32),
            pltpu.VMEM((CHH, CE), jnp.int32),
            pltpu.VMEM((CE,), jnp.float32),
            pltpu.VMEM((RPT,), jnp.float32),
            pltpu.VMEM_SHARED((NPAD,), jnp.float32),
        ],
    )


@functools.cache
def _build_prop():
    return pl.kernel(
        _prop_body,
        out_type=jax.ShapeDtypeStruct((NC, NPAD, DH), jnp.float32),
        mesh=_mesh(),
        compiler_params=pltpu.CompilerParams(use_tc_tiling_on_sc=False),
        scratch_types=[
            pltpu.VMEM((CH, CE), jnp.int32),
            pltpu.VMEM((CH, CE), jnp.int32),
        ] + [pltpu.VMEM((CE, DH), jnp.float32)] * NBUF + [
            pltpu.VMEM_SHARED((NPAD, DH), jnp.float32),
        ] + [pltpu.SemaphoreType.DMA] * (2 * NBUF),
    )


_R = 1024


def _dis_from(degT):
    deg = jnp.sum(degT, axis=1, keepdims=True)
    return jnp.where(deg > 0, lax.rsqrt(deg), 0.0)


def _tca_body(degT_ref, x_ref, w0_ref, w1_ref, g_ref, xw0_ref):
    dis = _dis_from(degT_ref[...])
    xb = x_ref[...]
    g = dis * jnp.dot(xb, w1_ref[...], preferred_element_type=jnp.float32)
    g_ref[0] = g[:, :DH]
    g_ref[1] = g[:, DH:]
    xw0_ref[...] = jnp.dot(xb, w0_ref[...], preferred_element_type=jnp.float32)


def _tcb_body(degT_ref, s_ref, xw0_ref, b_ref, w0_ref, w1_ref,
              g_ref, hw0_ref):
    dis = _dis_from(degT_ref[...])
    sfull = jnp.concatenate([s_ref[0], s_ref[1]], axis=1)
    h = xw0_ref[...] - dis * sfull + b_ref[...]
    h = jnp.maximum(h, 0.0)
    g = dis * jnp.dot(h, w1_ref[...], preferred_element_type=jnp.float32)
    g_ref[0] = g[:, :DH]
    g_ref[1] = g[:, DH:]
    hw0_ref[...] = jnp.dot(h, w0_ref[...], preferred_element_type=jnp.float32)


def _tcc_body(degT_ref, s_ref, hw0_ref, b_ref, out_ref):
    dis = _dis_from(degT_ref[...])
    sfull = jnp.concatenate([s_ref[0], s_ref[1]], axis=1)
    o = hw0_ref[...] - dis * sfull + b_ref[...]
    m = jnp.max(o, axis=1, keepdims=True)
    lse = jnp.log(jnp.sum(jnp.exp(o - m), axis=1, keepdims=True)) + m
    out_ref[...] = o - lse


def _row_spec(width):
    return pl.BlockSpec((_R, width), lambda i: (i, 0))


def _g_spec():
    return pl.BlockSpec((NC, _R, DH), lambda i: (0, i, 0))


def _full_spec(shape):
    return pl.BlockSpec(shape, lambda i: (0,) * len(shape))


_f32 = jnp.float32


def _tca(degT, x, w0, w1):
    return pl.pallas_call(
        _tca_body,
        grid=(NPAD // _R,),
        in_specs=[_row_spec(NC), _row_spec(D), _full_spec((D, D)),
                  _full_spec((D, D))],
        out_specs=[_g_spec(), _row_spec(D)],
        out_shape=[jax.ShapeDtypeStruct((NC, NPAD, DH), _f32),
                   jax.ShapeDtypeStruct((NPAD, D), _f32)],
    )(degT, x, w0, w1)


def _tcb(degT, s, xw0, b, w0, w1):
    return pl.pallas_call(
        _tcb_body,
        grid=(NPAD // _R,),
        in_specs=[_row_spec(NC), _g_spec(), _row_spec(D),
                  _full_spec((1, D)), _full_spec((D, D)), _full_spec((D, D))],
        out_specs=[_g_spec(), _row_spec(D)],
        out_shape=[jax.ShapeDtypeStruct((NC, NPAD, DH), _f32),
                   jax.ShapeDtypeStruct((NPAD, D), _f32)],
    )(degT, s, xw0, b, w0, w1)


def _tcc(degT, s, hw0, b):
    return pl.pallas_call(
        _tcc_body,
        grid=(NPAD // _R,),
        in_specs=[_row_spec(NC), _g_spec(), _row_spec(D),
                  _full_spec((1, D))],
        out_specs=_row_spec(D),
        out_shape=jax.ShapeDtypeStruct((N, D), _f32),
    )(degT, s, hw0, b)


def kernel(x, edge_index, W1, b1, W2, b2):
    row = edge_index[0]
    col = edge_index[1]
    pad = EPAD - E
    zpad = jnp.zeros((pad,), jnp.int32)
    row3 = jnp.concatenate([row, zpad]).reshape(NS, CH, CE)
    col3 = jnp.concatenate([col, zpad]).reshape(NS, CH, CE)
    x_p = jnp.pad(x, ((0, NPAD - N), (0, 0)))
    b1r = b1.reshape(1, D)
    b2r = b2.reshape(1, D)

    deg_call = _build_deg()
    prop_call = _build_prop()
    deg_parts, cfx3 = deg_call(row3, col3)
    degT = deg_parts.T
    g1, xw0 = _tca(degT, x_p, W1[0], W1[1])
    s1 = prop_call(g1, row3, cfx3)
    g2, hw0 = _tcb(degT, s1, xw0, b1r, W2[0], W2[1])
    s2 = prop_call(g2, row3, cfx3)
    return _tcc(degT, s2, hw0, b2r)


# speedup vs baseline: 11.4151x; 1.0249x over previous
import functools

import jax
import jax.numpy as jnp
from jax import lax
from jax.experimental import pallas as pl
from jax.experimental.pallas import tpu as pltpu
from jax.experimental.pallas import tpu_sc as plsc

N = 10000
D = 128
DH = D // 2
E = 320000
NC = 2
NS = 16
CE = 128
CH = 160
CHH = CH // NC
EPT = CH * CE
EPAD = NS * EPT
NPAD = 10240
RPT = NPAD // NS
TRASH = N


def _mesh():
    return plsc.VectorSubcoreMesh(core_axis_name="c", subcore_axis_name="s")


def _deg_body(row_hbm, col_hbm, out_hbm, cfx_hbm, ridx_v, col_v, ones_v,
              buf_v, hist_sh):
    c = lax.axis_index("c")
    s = lax.axis_index("s")
    base = c * CHH
    pltpu.sync_copy(row_hbm.at[s, pl.ds(base, CHH)], ridx_v)
    pltpu.sync_copy(col_hbm.at[s, pl.ds(base, CHH)], col_v)
    one = jnp.ones((16,), jnp.float32)
    zero = jnp.zeros((16,), jnp.float32)
    for k in range(CE // 16):
        ones_v[pl.ds(k * 16, 16)] = one

    def _remap(j, carry):
        for k in range(CE // 16):
            sl = pl.ds(k * 16, 16)
            r = ridx_v[j, sl]
            cc = col_v[j, sl]
            loop = r == cc
            ridx_v[j, sl] = jnp.where(loop, TRASH, r)
            col_v[j, sl] = jnp.where(loop, TRASH, cc)
        return carry

    lax.fori_loop(0, CHH, _remap, 0)
    pltpu.sync_copy(col_v, cfx_hbm.at[s, pl.ds(base, CHH)])

    def _zb(i, carry):
        buf_v[pl.ds(i * 16, 16)] = zero
        return carry

    lax.fori_loop(0, RPT // 16, _zb, 0)
    pltpu.sync_copy(buf_v, hist_sh.at[pl.ds(s * RPT, RPT)])
    plsc.subcore_barrier()

    def _acc(j, carry):
        pltpu.sync_copy(ones_v, hist_sh.at[ridx_v.at[j]], add=True)
        return carry

    lax.fori_loop(0, CHH, _acc, 0)
    plsc.subcore_barrier()
    pltpu.sync_copy(hist_sh.at[pl.ds(s * RPT, RPT)], buf_v)
    pltpu.sync_copy(buf_v, out_hbm.at[c, pl.ds(s * RPT, RPT)])


NBUF = 6


def _prop_body(g_hbm, row_hbm, cfx_hbm, out_hbm, row_v, cfx_v, *scr):
    bufs = scr[:NBUF]
    acc_sh = scr[NBUF]
    gsems = scr[NBUF + 1:2 * NBUF + 1]
    ssems = scr[2 * NBUF + 1:3 * NBUF + 1]
    rows0_v = bufs[0]
    c = lax.axis_index("c")
    s = lax.axis_index("s")
    pltpu.sync_copy(row_hbm.at[s], row_v)
    pltpu.sync_copy(cfx_hbm.at[s], cfx_v)

    zero = jnp.zeros((16,), jnp.float32)

    def _zb(i, carry):
        for k in range(DH // 16):
            rows0_v[i, pl.ds(k * 16, 16)] = zero
        return carry

    lax.fori_loop(0, CE, _zb, 0)
    for t in range(RPT // CE):
        pltpu.sync_copy(rows0_v, acc_sh.at[pl.ds(s * RPT + t * CE, CE)])

    gh = g_hbm.at[c]

    A = NBUF // 2
    for a in range(A):
        pltpu.async_copy(gh.at[row_v.at[a]], bufs[a], gsems[a])
    plsc.subcore_barrier()
    for j in range(CH):
        b = j % NBUF
        pltpu.make_async_copy(gh.at[row_v.at[j]], bufs[b], gsems[b]).wait()
        pltpu.async_copy(bufs[b], acc_sh.at[cfx_v.at[j]], ssems[b], add=True)
        jn = j + A
        if jn < CH:
            bn = jn % NBUF
            jp = jn - NBUF
            if jp >= 0:
                pltpu.make_async_copy(bufs[bn], acc_sh.at[cfx_v.at[jp]],
                                      ssems[bn]).wait()
            pltpu.async_copy(gh.at[row_v.at[jn]], bufs[bn], gsems[bn])
    for jj in range(max(CH - NBUF, 0), CH):
        pltpu.make_async_copy(bufs[jj % NBUF], acc_sh.at[cfx_v.at[jj]],
                              ssems[jj % NBUF]).wait()
    plsc.subcore_barrier()
    pltpu.sync_copy(acc_sh.at[pl.ds(s * RPT, RPT)], out_hbm.at[c, pl.ds(s * RPT, RPT)])


@functools.cache
def _build_deg():
    return pl.kernel(
        _deg_body,
        out_type=(jax.ShapeDtypeStruct((NC, NPAD), jnp.float32),
                  jax.ShapeDtypeStruct((NS, CH, CE), jnp.int32)),
        mesh=_mesh(),
        scratch_types=[
            pltpu.VMEM((CHH, CE), jnp.int32),
            pltpu.VMEM((CHH, CE), jnp.int32),
            pltpu.VMEM((CE,), jnp.float32),
            pltpu.VMEM((RPT,), jnp.float32),
            pltpu.VMEM_SHARED((NPAD,), jnp.float32),
        ],
    )


@functools.cache
def _build_prop():
    return pl.kernel(
        _prop_body,
        out_type=jax.ShapeDtypeStruct((NC, NPAD, DH), jnp.float32),
        mesh=_mesh(),
        compiler_params=pltpu.CompilerParams(use_tc_tiling_on_sc=False),
        scratch_types=[
            pltpu.VMEM((CH, CE), jnp.int32),
            pltpu.VMEM((CH, CE), jnp.int32),
        ] + [pltpu.VMEM((CE, DH), jnp.float32)] * NBUF + [
            pltpu.VMEM_SHARED((NPAD, DH), jnp.float32),
        ] + [pltpu.SemaphoreType.DMA] * (2 * NBUF),
    )


_R = 1024


def _dis_from(degT):
    deg = jnp.sum(degT, axis=1, keepdims=True)
    return jnp.where(deg > 0, lax.rsqrt(deg), 0.0)


def _tca_body(degT_ref, x_ref, w1_ref, g_ref):
    dis = _dis_from(degT_ref[...])
    g = dis * jnp.dot(x_ref[...], w1_ref[...],
                      preferred_element_type=jnp.float32)
    g_ref[0] = g[:, :DH]
    g_ref[1] = g[:, DH:]


def _tcmm_body(t_ref, w_ref, o_ref):
    o_ref[...] = jnp.dot(t_ref[...], w_ref[...],
                         preferred_element_type=jnp.float32)


def _tcb_body(degT_ref, s_ref, xw0_ref, b_ref, w1_ref, g_ref, h_ref):
    dis = _dis_from(degT_ref[...])
    sfull = jnp.concatenate([s_ref[0], s_ref[1]], axis=1)
    h = xw0_ref[...] - dis * sfull + b_ref[...]
    h = jnp.maximum(h, 0.0)
    g = dis * jnp.dot(h, w1_ref[...], preferred_element_type=jnp.float32)
    g_ref[0] = g[:, :DH]
    g_ref[1] = g[:, DH:]
    h_ref[...] = h


def _tcc_body(degT_ref, s_ref, hw0_ref, b_ref, out_ref):
    dis = _dis_from(degT_ref[...])
    sfull = jnp.concatenate([s_ref[0], s_ref[1]], axis=1)
    o = hw0_ref[...] - dis * sfull + b_ref[...]
    m = jnp.max(o, axis=1, keepdims=True)
    lse = jnp.log(jnp.sum(jnp.exp(o - m), axis=1, keepdims=True)) + m
    out_ref[...] = o - lse


def _row_spec(width):
    return pl.BlockSpec((_R, width), lambda i: (i, 0))


def _g_spec():
    return pl.BlockSpec((NC, _R, DH), lambda i: (0, i, 0))


def _full_spec(shape):
    return pl.BlockSpec(shape, lambda i: (0,) * len(shape))


_f32 = jnp.float32


def _tca(degT, x, w1):
    return pl.pallas_call(
        _tca_body,
        grid=(NPAD // _R,),
        in_specs=[_row_spec(NC), _row_spec(D), _full_spec((D, D))],
        out_specs=_g_spec(),
        out_shape=jax.ShapeDtypeStruct((NC, NPAD, DH), _f32),
    )(degT, x, w1)


def _tcmm(t, w):
    return pl.pallas_call(
        _tcmm_body,
        grid=(NPAD // _R,),
        in_specs=[_row_spec(D), _full_spec((D, D))],
        out_specs=_row_spec(D),
        out_shape=jax.ShapeDtypeStruct((NPAD, D), _f32),
    )(t, w)


def _tcb(degT, s, xw0, b, w1):
    return pl.pallas_call(
        _tcb_body,
        grid=(NPAD // _R,),
        in_specs=[_row_spec(NC), _g_spec(), _row_spec(D),
                  _full_spec((1, D)), _full_spec((D, D))],
        out_specs=[_g_spec(), _row_spec(D)],
        out_shape=[jax.ShapeDtypeStruct((NC, NPAD, DH), _f32),
                   jax.ShapeDtypeStruct((NPAD, D), _f32)],
    )(degT, s, xw0, b, w1)


def _tcc(degT, s, hw0, b):
    return pl.pallas_call(
        _tcc_body,
        grid=(NPAD // _R,),
        in_specs=[_row_spec(NC), _g_spec(), _row_spec(D),
                  _full_spec((1, D))],
        out_specs=_row_spec(D),
        out_shape=jax.ShapeDtypeStruct((N, D), _f32),
    )(degT, s, hw0, b)


def kernel(x, edge_index, W1, b1, W2, b2):
    row = edge_index[0]
    col = edge_index[1]
    pad = EPAD - E
    zpad = jnp.zeros((pad,), jnp.int32)
    row3 = jnp.concatenate([row, zpad]).reshape(NS, CH, CE)
    col3 = jnp.concatenate([col, zpad]).reshape(NS, CH, CE)
    x_p = jnp.pad(x, ((0, NPAD - N), (0, 0)))
    b1r = b1.reshape(1, D)
    b2r = b2.reshape(1, D)

    deg_call = _build_deg()
    prop_call = _build_prop()
    deg_parts, cfx3 = deg_call(row3, col3)
    degT = deg_parts.T
    g1 = _tca(degT, x_p, W1[1])
    s1 = prop_call(g1, row3, cfx3)
    xw0 = _tcmm(x_p, W1[0])
    g2, h = _tcb(degT, s1, xw0, b1r, W2[1])
    s2 = prop_call(g2, row3, cfx3)
    hw0 = _tcmm(h, W2[0])
    return _tcc(degT, s2, hw0, b2r)


# speedup vs baseline: 11.6333x; 1.0191x over previous
import functools

import jax
import jax.numpy as jnp
from jax import lax
from jax.experimental import pallas as pl
from jax.experimental.pallas import tpu as pltpu
from jax.experimental.pallas import tpu_sc as plsc

N = 10000
D = 128
DH = D // 2
E = 320000
NC = 2
NS = 16
CE = 128
CH = 160
CHH = CH // NC
EPT = CH * CE
EPAD = NS * EPT
NPAD = 10240
RPT = NPAD // NS
TRASH = N


def _mesh():
    return plsc.VectorSubcoreMesh(core_axis_name="c", subcore_axis_name="s")


def _deg_body(row_hbm, col_hbm, out_hbm, cfx_hbm, ridx_v, col_v, ones_v,
              buf_v, hist_sh):
    c = lax.axis_index("c")
    s = lax.axis_index("s")
    base = c * CHH
    pltpu.sync_copy(row_hbm.at[s, pl.ds(base, CHH)], ridx_v)
    pltpu.sync_copy(col_hbm.at[s, pl.ds(base, CHH)], col_v)
    one = jnp.ones((16,), jnp.float32)
    zero = jnp.zeros((16,), jnp.float32)
    for k in range(CE // 16):
        ones_v[pl.ds(k * 16, 16)] = one

    def _remap(j, carry):
        for k in range(CE // 16):
            sl = pl.ds(k * 16, 16)
            r = ridx_v[j, sl]
            cc = col_v[j, sl]
            loop = r == cc
            ridx_v[j, sl] = jnp.where(loop, TRASH, r)
            col_v[j, sl] = jnp.where(loop, TRASH, cc)
        return carry

    lax.fori_loop(0, CHH, _remap, 0)
    pltpu.sync_copy(col_v, cfx_hbm.at[s, pl.ds(base, CHH)])

    def _zb(i, carry):
        buf_v[pl.ds(i * 16, 16)] = zero
        return carry

    lax.fori_loop(0, RPT // 16, _zb, 0)
    pltpu.sync_copy(buf_v, hist_sh.at[pl.ds(s * RPT, RPT)])
    plsc.subcore_barrier()

    def _acc(j, carry):
        pltpu.sync_copy(ones_v, hist_sh.at[ridx_v.at[j]], add=True)
        return carry

    lax.fori_loop(0, CHH, _acc, 0)
    plsc.subcore_barrier()
    pltpu.sync_copy(hist_sh.at[pl.ds(s * RPT, RPT)], buf_v)
    pltpu.sync_copy(buf_v, out_hbm.at[c, pl.ds(s * RPT, RPT)])


NBUF = 6


def _prop_body(g_hbm, row_hbm, cfx_hbm, out_hbm, row_v, cfx_v, *scr):
    bufs = scr[:NBUF]
    acc_sh = scr[NBUF]
    gsems = scr[NBUF + 1:2 * NBUF + 1]
    ssems = scr[2 * NBUF + 1:3 * NBUF + 1]
    rows0_v = bufs[0]
    c = lax.axis_index("c")
    s = lax.axis_index("s")
    pltpu.sync_copy(row_hbm.at[s], row_v)
    pltpu.sync_copy(cfx_hbm.at[s], cfx_v)

    zero = jnp.zeros((16,), jnp.float32)

    def _zb(i, carry):
        for k in range(DH // 16):
            rows0_v[i, pl.ds(k * 16, 16)] = zero
        return carry

    lax.fori_loop(0, CE, _zb, 0)
    for t in range(RPT // CE):
        pltpu.sync_copy(rows0_v, acc_sh.at[pl.ds(s * RPT + t * CE, CE)])

    gh = g_hbm.at[c]

    A = 4
    for a in range(A):
        pltpu.async_copy(gh.at[row_v.at[a]], bufs[a], gsems[a])
    plsc.subcore_barrier()
    for j in range(CH):
        b = j % NBUF
        pltpu.make_async_copy(gh.at[row_v.at[j]], bufs[b], gsems[b]).wait()
        pltpu.async_copy(bufs[b], acc_sh.at[cfx_v.at[j]], ssems[b], add=True)
        jn = j + A
        if jn < CH:
            bn = jn % NBUF
            jp = jn - NBUF
            if jp >= 0:
                pltpu.make_async_copy(bufs[bn], acc_sh.at[cfx_v.at[jp]],
                                      ssems[bn]).wait()
            pltpu.async_copy(gh.at[row_v.at[jn]], bufs[bn], gsems[bn])
    for jj in range(max(CH - NBUF, 0), CH):
        pltpu.make_async_copy(bufs[jj % NBUF], acc_sh.at[cfx_v.at[jj]],
                              ssems[jj % NBUF]).wait()
    plsc.subcore_barrier()
    pltpu.sync_copy(acc_sh.at[pl.ds(s * RPT, RPT)], out_hbm.at[c, pl.ds(s * RPT, RPT)])


@functools.cache
def _build_deg():
    return pl.kernel(
        _deg_body,
        out_type=(jax.ShapeDtypeStruct((NC, NPAD), jnp.float32),
                  jax.ShapeDtypeStruct((NS, CH, CE), jnp.int32)),
        mesh=_mesh(),
        scratch_types=[
            pltpu.VMEM((CHH, CE), jnp.int32),
            pltpu.VMEM((CHH, CE), jnp.int32),
            pltpu.VMEM((CE,), jnp.float32),
            pltpu.VMEM((RPT,), jnp.float32),
            pltpu.VMEM_SHARED((NPAD,), jnp.float32),
        ],
    )


@functools.cache
def _build_prop():
    return pl.kernel(
        _prop_body,
        out_type=jax.ShapeDtypeStruct((NC, NPAD, DH), jnp.float32),
        mesh=_mesh(),
        compiler_params=pltpu.CompilerParams(use_tc_tiling_on_sc=False),
        scratch_types=[
            pltpu.VMEM((CH, CE), jnp.int32),
            pltpu.VMEM((CH, CE), jnp.int32),
        ] + [pltpu.VMEM((CE, DH), jnp.float32)] * NBUF + [
            pltpu.VMEM_SHARED((NPAD, DH), jnp.float32),
        ] + [pltpu.SemaphoreType.DMA] * (2 * NBUF),
    )


_R = 1024


def _dis_from(degT):
    deg = jnp.sum(degT, axis=1, keepdims=True)
    return jnp.where(deg > 0, lax.rsqrt(deg), 0.0)


def _tca_body(degT_ref, x_ref, w1_ref, g_ref):
    dis = _dis_from(degT_ref[...])
    g = dis * jnp.dot(x_ref[...], w1_ref[...],
                      preferred_element_type=jnp.float32)
    g_ref[0] = g[:, :DH]
    g_ref[1] = g[:, DH:]


def _tcmm_body(t_ref, w_ref, o_ref):
    o_ref[...] = jnp.dot(t_ref[...], w_ref[...],
                         preferred_element_type=jnp.float32)


def _tcb_body(degT_ref, s_ref, xw0_ref, b_ref, w1_ref, g_ref, h_ref):
    dis = _dis_from(degT_ref[...])
    sfull = jnp.concatenate([s_ref[0], s_ref[1]], axis=1)
    h = xw0_ref[...] - dis * sfull + b_ref[...]
    h = jnp.maximum(h, 0.0)
    g = dis * jnp.dot(h, w1_ref[...], preferred_element_type=jnp.float32)
    g_ref[0] = g[:, :DH]
    g_ref[1] = g[:, DH:]
    h_ref[...] = h


def _tcc_body(degT_ref, s_ref, hw0_ref, b_ref, out_ref):
    dis = _dis_from(degT_ref[...])
    sfull = jnp.concatenate([s_ref[0], s_ref[1]], axis=1)
    o = hw0_ref[...] - dis * sfull + b_ref[...]
    m = jnp.max(o, axis=1, keepdims=True)
    lse = jnp.log(jnp.sum(jnp.exp(o - m), axis=1, keepdims=True)) + m
    out_ref[...] = o - lse


def _row_spec(width):
    return pl.BlockSpec((_R, width), lambda i: (i, 0))


def _g_spec():
    return pl.BlockSpec((NC, _R, DH), lambda i: (0, i, 0))


def _full_spec(shape):
    return pl.BlockSpec(shape, lambda i: (0,) * len(shape))


_f32 = jnp.float32


def _tca(degT, x, w1):
    return pl.pallas_call(
        _tca_body,
        grid=(NPAD // _R,),
        in_specs=[_row_spec(NC), _row_spec(D), _full_spec((D, D))],
        out_specs=_g_spec(),
        out_shape=jax.ShapeDtypeStruct((NC, NPAD, DH), _f32),
    )(degT, x, w1)


def _tcmm(t, w):
    return pl.pallas_call(
        _tcmm_body,
        grid=(NPAD // _R,),
        in_specs=[_row_spec(D), _full_spec((D, D))],
        out_specs=_row_spec(D),
        out_shape=jax.ShapeDtypeStruct((NPAD, D), _f32),
    )(t, w)


def _tcb(degT, s, xw0, b, w1):
    return pl.pallas_call(
        _tcb_body,
        grid=(NPAD // _R,),
        in_specs=[_row_spec(NC), _g_spec(), _row_spec(D),
                  _full_spec((1, D)), _full_spec((D, D))],
        out_specs=[_g_spec(), _row_spec(D)],
        out_shape=[jax.ShapeDtypeStruct((NC, NPAD, DH), _f32),
                   jax.ShapeDtypeStruct((NPAD, D), _f32)],
    )(degT, s, xw0, b, w1)


def _tcc(degT, s, hw0, b):
    return pl.pallas_call(
        _tcc_body,
        grid=(NPAD // _R,),
        in_specs=[_row_spec(NC), _g_spec(), _row_spec(D),
                  _full_spec((1, D))],
        out_specs=_row_spec(D),
        out_shape=jax.ShapeDtypeStruct((N, D), _f32),
    )(degT, s, hw0, b)


def kernel(x, edge_index, W1, b1, W2, b2):
    row = edge_index[0]
    col = edge_index[1]
    pad = EPAD - E
    zpad = jnp.zeros((pad,), jnp.int32)
    row3 = jnp.concatenate([row, zpad]).reshape(NS, CH, CE)
    col3 = jnp.concatenate([col, zpad]).reshape(NS, CH, CE)
    x_p = jnp.pad(x, ((0, NPAD - N), (0, 0)))
    b1r = b1.reshape(1, D)
    b2r = b2.reshape(1, D)

    deg_call = _build_deg()
    prop_call = _build_prop()
    deg_parts, cfx3 = deg_call(row3, col3)
    degT = deg_parts.T
    g1 = _tca(degT, x_p, W1[1])
    s1 = prop_call(g1, row3, cfx3)
    xw0 = _tcmm(x_p, W1[0])
    g2, h = _tcb(degT, s1, xw0, b1r, W2[1])
    s2 = prop_call(g2, row3, cfx3)
    hw0 = _tcmm(h, W2[0])
    return _tcc(degT, s2, hw0, b2r)
